# baseline (speedup 1.0000x reference)
import functools

import jax
import jax.numpy as jnp
from jax import lax
from jax.experimental import pallas as pl
from jax.experimental.pallas import tpu as pltpu
from jax.experimental.pallas import tpu_sc as plsc

S = 512
D = 512
NC = 2
NS = 16
NW = NC * NS
T = 8
BJ = 64
WROWS = 8 * (T - 1) + BJ
NITEM = 16

_mesh = plsc.VectorSubcoreMesh(core_axis_name="c", subcore_axis_name="s")


@functools.partial(
    pl.kernel,
    mesh=_mesh,
    out_type=jax.ShapeDtypeStruct((S * S, D), jnp.float32),
    scratch_types=[
        pltpu.VMEM((128,), jnp.int32),
        pltpu.VMEM((128,), jnp.int32),
        pltpu.VMEM((WROWS, D), jnp.float32),
        pltpu.VMEM((WROWS, D), jnp.float32),
        pltpu.SemaphoreType.DMA,
        pltpu.SemaphoreType.DMA,
        pltpu.SemaphoreType.DMA,
        pltpu.SemaphoreType.DMA,
    ],
)
def _expand(relp_hbm, out_hbm, idx0, idx1, win0, win1, sr0, sr1, sw0, sw1):
    idxs, wins = (idx0, idx1), (win0, win1)
    srs, sws = (sr0, sr1), (sw0, sw1)
    wid = lax.axis_index("s") * NC + lax.axis_index("c")
    lane = lax.broadcasted_iota(jnp.int32, (16,), 0)

    def decode(k):
        combo = wid * NITEM + k
        m = combo % 8
        p = (combo // 8) % 8
        jb = combo // 64
        ibase = m + 64 * p
        j0 = jb * BJ
        w0 = (S - 1) - (ibase + 8 * (T - 1)) + j0
        return ibase, j0, (2 * S - 1) - w0

    def start_gather(k):
        b = k % 2
        _, _, top = decode(k)
        for u in range(WROWS // 16 + 1):
            idxs[b][pl.ds(u * 16, 16)] = (top - u * 16) - lane
        pltpu.make_async_copy(
            relp_hbm.at[idxs[b].at[pl.ds(0, WROWS)]], wins[b], srs[b]).start()

    start_gather(0)
    for k in range(NITEM):
        b = k % 2
        if k > 0:
            for _ in range(T):
                pltpu.make_async_copy(
                    wins[1 - b].at[pl.ds(0, BJ)],
                    out_hbm.at[pl.ds(0, BJ)], sws[1 - b]).wait()
        if k + 1 < NITEM:
            start_gather(k + 1)
        pltpu.make_async_copy(
            relp_hbm.at[idxs[b].at[pl.ds(0, WROWS)]], wins[b], srs[b]).wait()
        ibase, j0, _ = decode(k)
        for t in range(T):
            pltpu.make_async_copy(
                wins[b].at[pl.ds(8 * (T - 1 - t), BJ)],
                out_hbm.at[pl.ds((ibase + 8 * t) * S + j0, BJ)],
                sws[b]).start()
    for _ in range(T):
        pltpu.make_async_copy(
            wins[1].at[pl.ds(0, BJ)], out_hbm.at[pl.ds(0, BJ)], sws[1]).wait()


def kernel(seq_len, rel_pos_emb):
    del seq_len
    max_len = (rel_pos_emb.shape[0] + 1) // 2
    start = max_len - 1 - (S - 1)
    relp = lax.slice_in_dim(rel_pos_emb, start - 1, start + 2 * S - 1, axis=0)
    out_flat = _expand(relp)
    return out_flat.reshape(S, S, D)


# speedup vs baseline: 1.2168x; 1.2168x over previous
import functools

import jax
import jax.numpy as jnp
from jax import lax
from jax.experimental import pallas as pl
from jax.experimental.pallas import tpu as pltpu
from jax.experimental.pallas import tpu_sc as plsc

S = 512
D = 512
NC = 2
NS = 16
NW = NC * NS
T = 16
BJ = 128
WROWS = 8 * (T - 1) + BJ
NITEM = 4

_mesh = plsc.VectorSubcoreMesh(core_axis_name="c", subcore_axis_name="s")


@functools.partial(
    pl.kernel,
    mesh=_mesh,
    out_type=jax.ShapeDtypeStruct((S * S, D), jnp.float32),
    scratch_types=[
        pltpu.VMEM((2 * T * 16,), jnp.int32),
        pltpu.VMEM((WROWS, D), jnp.float32),
        pltpu.SemaphoreType.DMA,
        pltpu.SemaphoreType.DMA,
        pltpu.SemaphoreType.DMA,
    ],
)
def _expand(tab_hbm, out_hbm, idx_v, win_v, sem_r1, sem_r2, sem_w):
    wid = lax.axis_index("s") * NC + lax.axis_index("c")
    lane = lax.broadcasted_iota(jnp.int32, (16,), 0)
    top0 = (tab_hbm.shape[0] + 1) // 2 - 1 + (S - 1)

    for cc in range(NITEM):
        combo = wid * NITEM + cc
        m = combo % 8
        p = (combo // 8) % 4
        jb = combo // 32
        ibase = m + 128 * p
        j0 = jb * BJ
        w0 = (S - 1) - (ibase + 8 * (T - 1)) + j0
        top = top0 - w0
        for u in range(WROWS // 16 + 1):
            idx_v[pl.ds(u * 16, 16)] = (top - u * 16) - lane
        pltpu.make_async_copy(
            tab_hbm.at[idx_v.at[pl.ds(0, 128)]],
            win_v.at[pl.ds(0, 128)], sem_r1).start()
        pltpu.make_async_copy(
            tab_hbm.at[idx_v.at[pl.ds(128, WROWS - 128)]],
            win_v.at[pl.ds(128, WROWS - 128)], sem_r2).start()

        pltpu.make_async_copy(
            tab_hbm.at[idx_v.at[pl.ds(0, 128)]],
            win_v.at[pl.ds(0, 128)], sem_r1).wait()
        pltpu.make_async_copy(
            win_v.at[pl.ds(0, BJ)],
            out_hbm.at[pl.ds((ibase + 8 * (T - 1)) * S + j0, BJ)],
            sem_w).start()
        pltpu.make_async_copy(
            tab_hbm.at[idx_v.at[pl.ds(128, WROWS - 128)]],
            win_v.at[pl.ds(128, WROWS - 128)], sem_r2).wait()
        for t in range(T - 1):
            pltpu.make_async_copy(
                win_v.at[pl.ds(8 * (T - 1 - t), BJ)],
                out_hbm.at[pl.ds((ibase + 8 * t) * S + j0, BJ)],
                sem_w).start()
        for _ in range(T):
            pltpu.make_async_copy(
                win_v.at[pl.ds(0, BJ)],
                out_hbm.at[pl.ds(0, BJ)], sem_w).wait()


def kernel(seq_len, rel_pos_emb):
    del seq_len
    out_flat = _expand(rel_pos_emb)
    return out_flat.reshape(S, S, D)


# speedup vs baseline: 1.2216x; 1.0040x over previous
import functools

import jax
import jax.numpy as jnp
from jax import lax
from jax.experimental import pallas as pl
from jax.experimental.pallas import tpu as pltpu
from jax.experimental.pallas import tpu_sc as plsc

S = 512
D = 512
NC = 2
NS = 16
NW = NC * NS
T = 16
BJ = 128
WROWS = 8 * (T - 1) + BJ
NITEM = 4

_mesh = plsc.VectorSubcoreMesh(core_axis_name="c", subcore_axis_name="s")


@functools.partial(
    pl.kernel,
    mesh=_mesh,
    out_type=jax.ShapeDtypeStruct((S * S, D), jnp.float32),
    scratch_types=[
        pltpu.VMEM((2 * T * 16,), jnp.int32),
        pltpu.VMEM((WROWS, D), jnp.float32),
        pltpu.SemaphoreType.DMA,
        pltpu.SemaphoreType.DMA,
        pltpu.SemaphoreType.DMA,
        pltpu.SemaphoreType.DMA,
    ],
)
def _expand(tab_hbm, out_hbm, idx_v, win_v, sem_r1, sem_r2, sem_r3, sem_w):
    wid = lax.axis_index("s") * NC + lax.axis_index("c")
    lane = lax.broadcasted_iota(jnp.int32, (16,), 0)
    top0 = (tab_hbm.shape[0] + 1) // 2 - 1 + (S - 1)

    for cc in range(NITEM):
        combo = wid * NITEM + cc
        m = combo % 8
        p = (combo // 8) % 4
        jb = combo // 32
        ibase = m + 128 * p
        j0 = jb * BJ
        w0 = (S - 1) - (ibase + 8 * (T - 1)) + j0
        top = top0 - w0
        for u in range(WROWS // 16 + 1):
            idx_v[pl.ds(u * 16, 16)] = (top - u * 16) - lane
        pieces = ((0, 128, sem_r1), (128, 48, sem_r2), (176, 72, sem_r3))
        for off, ln, sem in pieces:
            pltpu.make_async_copy(
                tab_hbm.at[idx_v.at[pl.ds(off, ln)]],
                win_v.at[pl.ds(off, ln)], sem).start()

        def fire(t):
            pltpu.make_async_copy(
                win_v.at[pl.ds(8 * (T - 1 - t), BJ)],
                out_hbm.at[pl.ds((ibase + 8 * t) * S + j0, BJ)],
                sem_w).start()

        unlocked = (range(T - 1, T), range(9, T - 1), range(0, 9))
        for (off, ln, sem), ts in zip(pieces, unlocked):
            pltpu.make_async_copy(
                tab_hbm.at[idx_v.at[pl.ds(off, ln)]],
                win_v.at[pl.ds(off, ln)], sem).wait()
            for t in ts:
                fire(t)
        for _ in range(T):
            pltpu.make_async_copy(
                win_v.at[pl.ds(0, BJ)],
                out_hbm.at[pl.ds(0, BJ)], sem_w).wait()


def kernel(seq_len, rel_pos_emb):
    del seq_len
    out_flat = _expand(rel_pos_emb)
    return out_flat.reshape(S, S, D)
